# baseline (speedup 1.0000x reference)
import jax
import jax.numpy as jnp
from jax.experimental import pallas as pl
from jax.experimental.pallas import tpu as pltpu

B, L, CLEN, E, TOP_K, D_MODEL = 256, 8, 128, 64, 2, 128
DIN = 3 * CLEN
R = B * L
EPS = 1e-9

GB = 8
NCHUNK = E // GB

_DN_T = (((0,), (0,)), ((), ()))


def _routing_T(logitsT, masksT):
    maskf = (masksT == 1).astype(jnp.float32)
    colmax = jnp.max(logitsT, axis=0, keepdims=True)
    ex = jnp.exp(logitsT - colmax)
    g = ex / jnp.sum(ex, axis=0, keepdims=True) * maskf

    iota = jax.lax.broadcasted_iota(jnp.int32, (E, B), 0)
    v1 = jnp.max(g, axis=0, keepdims=True)
    idx1 = jnp.min(jnp.where(g == v1, iota, E), axis=0, keepdims=True)
    oh1 = iota == idx1
    g2 = jnp.where(oh1, -1.0, g)
    v2 = jnp.max(g2, axis=0, keepdims=True)
    idx2 = jnp.min(jnp.where(g2 == v2, iota, E), axis=0, keepdims=True)
    oh2 = iota == idx2
    denom = v1 + v2 + EPS
    return (jnp.where(oh1, v1, 0.0) + jnp.where(oh2, v2, 0.0)) / denom


def _moe_kernel(logitsT_ref, masksT_ref, flat_hbm_ref, W_hbm_ref, b_ref,
                out_ref, wbuf_ref, flat_ref, dma_sem, flat_sem):
    def _copy(chunk):
        return pltpu.make_async_copy(
            W_hbm_ref.at[pl.ds(chunk * GB, GB)],
            wbuf_ref.at[chunk % 2],
            dma_sem.at[chunk % 2],
        )

    flat_copy = pltpu.make_async_copy(flat_hbm_ref, flat_ref, flat_sem)
    flat_copy.start()
    _copy(0).start()
    _copy(1).start()

    cT = _routing_T(logitsT_ref[...], masksT_ref[...])
    lane_b = jax.lax.broadcasted_iota(jnp.int32, (B, R), 1) // L
    sub_b = jax.lax.broadcasted_iota(jnp.int32, (B, R), 0)
    exp_mat = (lane_b == sub_b).astype(jnp.bfloat16)
    c2T = jnp.dot(cT.astype(jnp.bfloat16), exp_mat,
                  preferred_element_type=jnp.float32)

    flat_copy.wait()
    xT = flat_ref[...].T.astype(jnp.bfloat16)
    acc = jax.lax.dot_general(
        b_ref[...], c2T, _DN_T, preferred_element_type=jnp.float32)

    for g in range(NCHUNK):
        slot = g % 2
        _copy(g).wait()
        for j in range(GB):
            e = g * GB + j
            w = wbuf_ref[slot, j].astype(jnp.bfloat16)
            y = jax.lax.dot_general(w, xT, _DN_T,
                                    preferred_element_type=jnp.float32)
            acc = acc + c2T[e:e + 1, :] * y
        if g + 2 < NCHUNK:
            _copy(g + 2).start()
    out_ref[...] = acc.astype(jnp.bfloat16).T


def kernel(cycle_curve_data, logits, moe_masks, W, b):
    flat2 = cycle_curve_data.reshape(R, DIN)
    W_hbm = pltpu.with_memory_space_constraint(W, pltpu.MemorySpace.HBM)
    flat_hbm = pltpu.with_memory_space_constraint(flat2, pltpu.MemorySpace.HBM)
    out = pl.pallas_call(
        _moe_kernel,
        in_specs=[
            pl.BlockSpec((E, B), lambda: (0, 0)),
            pl.BlockSpec((E, B), lambda: (0, 0)),
            pl.BlockSpec(memory_space=pltpu.MemorySpace.HBM),
            pl.BlockSpec(memory_space=pltpu.MemorySpace.HBM),
            pl.BlockSpec((E, D_MODEL), lambda: (0, 0)),
        ],
        out_specs=pl.BlockSpec((R, D_MODEL), lambda: (0, 0)),
        out_shape=jax.ShapeDtypeStruct((R, D_MODEL), jnp.bfloat16),
        scratch_shapes=[
            pltpu.VMEM((2, GB, DIN, D_MODEL), jnp.float32),
            pltpu.VMEM((R, DIN), jnp.float32),
            pltpu.SemaphoreType.DMA((2,)),
            pltpu.SemaphoreType.DMA,
        ],
    )(logits.T, moe_masks.T, flat_hbm, W_hbm, b)
    return out.reshape(B, L, D_MODEL)


# speedup vs baseline: 1.0151x; 1.0151x over previous
import jax
import jax.numpy as jnp
from jax.experimental import pallas as pl
from jax.experimental.pallas import tpu as pltpu

B, L, CLEN, E, TOP_K, D_MODEL = 256, 8, 128, 64, 2, 128
DIN = 3 * CLEN
R = B * L
EPS = 1e-9

GB = 8
NCHUNK = E // GB

_DN_T = (((0,), (0,)), ((), ()))


def _routing_T(logitsT, masksT):
    maskf = (masksT == 1).astype(jnp.float32)
    colmax = jnp.max(logitsT, axis=0, keepdims=True)
    ex = jnp.exp(logitsT - colmax)
    g = ex / jnp.sum(ex, axis=0, keepdims=True) * maskf

    iota = jax.lax.broadcasted_iota(jnp.int32, (E, B), 0)
    v1 = jnp.max(g, axis=0, keepdims=True)
    idx1 = jnp.min(jnp.where(g == v1, iota, E), axis=0, keepdims=True)
    oh1 = iota == idx1
    g2 = jnp.where(oh1, -1.0, g)
    v2 = jnp.max(g2, axis=0, keepdims=True)
    idx2 = jnp.min(jnp.where(g2 == v2, iota, E), axis=0, keepdims=True)
    oh2 = iota == idx2
    denom = v1 + v2 + EPS
    return (jnp.where(oh1, v1, 0.0) + jnp.where(oh2, v2, 0.0)) / denom


def _moe_kernel(logitsT_ref, masksT_ref, flat_ref, W_hbm_ref, b_ref,
                out_ref, wbuf_ref, dma_sem):
    def _copy(chunk):
        return pltpu.make_async_copy(
            W_hbm_ref.at[pl.ds(chunk * GB, GB)],
            wbuf_ref.at[chunk % 2],
            dma_sem.at[chunk % 2],
        )

    _copy(0).start()
    _copy(1).start()

    cT = _routing_T(logitsT_ref[...], masksT_ref[...])
    lane_b = jax.lax.broadcasted_iota(jnp.int32, (B, R), 1) // L
    sub_b = jax.lax.broadcasted_iota(jnp.int32, (B, R), 0)
    exp_mat = (lane_b == sub_b).astype(jnp.bfloat16)
    c2T = jnp.dot(cT.astype(jnp.bfloat16), exp_mat,
                  preferred_element_type=jnp.float32)

    xT = flat_ref[...].T.astype(jnp.bfloat16)
    acc = jax.lax.dot_general(
        b_ref[...], c2T, _DN_T, preferred_element_type=jnp.float32)

    for g in range(NCHUNK):
        slot = g % 2
        _copy(g).wait()
        for j in range(GB):
            e = g * GB + j
            w = wbuf_ref[slot, j].astype(jnp.bfloat16)
            y = jax.lax.dot_general(w, xT, _DN_T,
                                    preferred_element_type=jnp.float32)
            acc = acc + c2T[e:e + 1, :] * y
        if g + 2 < NCHUNK:
            _copy(g + 2).start()
    out_ref[...] = acc.astype(jnp.bfloat16).T


def kernel(cycle_curve_data, logits, moe_masks, W, b):
    flat2 = cycle_curve_data.reshape(R, DIN)
    W_hbm = pltpu.with_memory_space_constraint(W, pltpu.MemorySpace.HBM)
    out = pl.pallas_call(
        _moe_kernel,
        in_specs=[
            pl.BlockSpec((E, B), lambda: (0, 0)),
            pl.BlockSpec((E, B), lambda: (0, 0)),
            pl.BlockSpec((R, DIN), lambda: (0, 0)),
            pl.BlockSpec(memory_space=pltpu.MemorySpace.HBM),
            pl.BlockSpec((E, D_MODEL), lambda: (0, 0)),
        ],
        out_specs=pl.BlockSpec((R, D_MODEL), lambda: (0, 0)),
        out_shape=jax.ShapeDtypeStruct((R, D_MODEL), jnp.bfloat16),
        scratch_shapes=[
            pltpu.VMEM((2, GB, DIN, D_MODEL), jnp.float32),
            pltpu.SemaphoreType.DMA((2,)),
        ],
    )(logits.T, moe_masks.T, flat2, W_hbm, b)
    return out.reshape(B, L, D_MODEL)
